# baseline (speedup 1.0000x reference)
import functools

import jax
import jax.numpy as jnp
from jax.experimental import pallas as pl
from jax.experimental.pallas import tpu as pltpu

_NUM_EXPERTS = 8
_NUM_TOPICS = 4
_TOP_K = 2
_D_MODEL = 768
_N_TOKENS = 32768

_BLOCK = 4096


def _fused_body(u_ref, wg_ref, bg_ref, wh_ref, bh_ref,
                out_ref, idx_ref, aux_ref, imp_ref, load_ref):
    i = pl.program_id(0)
    nsteps = pl.num_programs(0)

    @pl.when(i == 0)
    def _init():
        imp_ref[...] = jnp.zeros_like(imp_ref)
        load_ref[...] = jnp.zeros_like(load_ref)

    logits = jnp.dot(u_ref[...], wg_ref[...],
                     preferred_element_type=jnp.float32) + bg_ref[...]
    m = jnp.max(logits, axis=-1, keepdims=True)
    e = jnp.exp(logits - m)
    denom = jnp.sum(e, axis=-1, keepdims=True)
    all_s = e / denom

    col = jax.lax.broadcasted_iota(jnp.int32, all_s.shape, 1)
    m1 = jnp.max(all_s, axis=-1, keepdims=True)
    a1 = jnp.min(jnp.where(all_s == m1, col, _NUM_EXPERTS),
                 axis=-1, keepdims=True)
    masked = jnp.where(col == a1, -jnp.inf, all_s)
    m2 = jnp.max(masked, axis=-1, keepdims=True)
    a2 = jnp.min(jnp.where(masked == m2, col, _NUM_EXPERTS),
                 axis=-1, keepdims=True)
    idx_ref[...] = jnp.concatenate([a1, a2], axis=1)

    imp_ref[...] += jnp.sum(all_s, axis=0, keepdims=True)
    onehot = ((col == a1) | (col == a2)).astype(jnp.float32)
    load_ref[...] += jnp.sum(onehot, axis=0, keepdims=True)

    out_ref[...] = jnp.dot(all_s, wh_ref[...],
                           preferred_element_type=jnp.float32) + bh_ref[...]

    @pl.when(i == nsteps - 1)
    def _finish():
        scale = _NUM_EXPERTS / (float(_N_TOKENS) * float(_N_TOKENS))
        aux_ref[...] = scale * jnp.sum(imp_ref[...] * load_ref[...],
                                       keepdims=True)


@functools.partial(jax.jit, static_argnames=())
def kernel(u, W_g, b_g, W_h, b_h):
    n, d = u.shape
    nblk = n // _BLOCK
    out, idx, aux = pl.pallas_call(
        _fused_body,
        grid=(nblk,),
        in_specs=[
            pl.BlockSpec((_BLOCK, d), lambda i: (i, 0)),
            pl.BlockSpec((d, _NUM_EXPERTS), lambda i: (0, 0)),
            pl.BlockSpec((1, _NUM_EXPERTS), lambda i: (0, 0)),
            pl.BlockSpec((_NUM_EXPERTS, _NUM_TOPICS), lambda i: (0, 0)),
            pl.BlockSpec((1, _NUM_TOPICS), lambda i: (0, 0)),
        ],
        out_specs=[
            pl.BlockSpec((_BLOCK, _NUM_TOPICS), lambda i: (i, 0)),
            pl.BlockSpec((_BLOCK, _TOP_K), lambda i: (i, 0)),
            pl.BlockSpec((1, 1), lambda i: (0, 0)),
        ],
        out_shape=[
            jax.ShapeDtypeStruct((n, _NUM_TOPICS), jnp.float32),
            jax.ShapeDtypeStruct((n, _TOP_K), jnp.int32),
            jax.ShapeDtypeStruct((1, 1), jnp.float32),
        ],
        scratch_shapes=[
            pltpu.VMEM((1, _NUM_EXPERTS), jnp.float32),
            pltpu.VMEM((1, _NUM_EXPERTS), jnp.float32),
        ],
    )(u, W_g, b_g.reshape(1, -1), W_h, b_h.reshape(1, -1))
    return (out, aux[0, 0], idx)


# speedup vs baseline: 2.0999x; 2.0999x over previous
import functools

import jax
import jax.numpy as jnp
from jax.experimental import pallas as pl
from jax.experimental.pallas import tpu as pltpu

_NUM_EXPERTS = 8
_NUM_TOPICS = 4
_TOP_K = 2
_N_TOKENS = 32768

_BLOCK = 4096


def _fused_body(u_ref, wgt_ref, bg_ref, wht_ref, bh_ref,
                out_ref, idx_ref, aux_ref, imp_ref, load_ref):
    i = pl.program_id(0)
    nsteps = pl.num_programs(0)

    @pl.when(i == 0)
    def _init():
        imp_ref[...] = jnp.zeros_like(imp_ref)
        load_ref[...] = jnp.zeros_like(load_ref)

    logits = jax.lax.dot_general(
        wgt_ref[...], u_ref[...],
        dimension_numbers=(((1,), (1,)), ((), ())),
        preferred_element_type=jnp.float32) + bg_ref[...]

    m = jnp.max(logits, axis=0, keepdims=True)
    e = jnp.exp(logits - m)
    denom = jnp.sum(e, axis=0, keepdims=True)
    all_s = e * (1.0 / denom)

    row = jax.lax.broadcasted_iota(jnp.int32, all_s.shape, 0)
    m1 = jnp.max(all_s, axis=0, keepdims=True)
    a1 = jnp.min(jnp.where(all_s == m1, row, _NUM_EXPERTS),
                 axis=0, keepdims=True)
    masked = jnp.where(row == a1, -jnp.inf, all_s)
    m2 = jnp.max(masked, axis=0, keepdims=True)
    a2 = jnp.min(jnp.where(masked == m2, row, _NUM_EXPERTS),
                 axis=0, keepdims=True)
    idx_ref[...] = jnp.concatenate([a1, a2], axis=0)

    imp_ref[...] += jnp.sum(all_s, axis=1, keepdims=True)
    onehot = ((row == a1) | (row == a2)).astype(jnp.float32)
    load_ref[...] += jnp.sum(onehot, axis=1, keepdims=True)

    out_ref[...] = jax.lax.dot_general(
        wht_ref[...], all_s,
        dimension_numbers=(((1,), (0,)), ((), ())),
        preferred_element_type=jnp.float32) + bh_ref[...]

    @pl.when(i == nsteps - 1)
    def _finish():
        scale = _NUM_EXPERTS / (float(_N_TOKENS) * float(_N_TOKENS))
        aux_ref[...] = scale * jnp.sum(imp_ref[...] * load_ref[...],
                                       keepdims=True)


@functools.partial(jax.jit, static_argnames=())
def kernel(u, W_g, b_g, W_h, b_h):
    n, d = u.shape
    nblk = n // _BLOCK
    n_exp = W_g.shape[1]
    n_top = W_h.shape[1]
    out_t, idx_t, aux = pl.pallas_call(
        _fused_body,
        grid=(nblk,),
        in_specs=[
            pl.BlockSpec((_BLOCK, d), lambda i: (i, 0)),
            pl.BlockSpec((n_exp, d), lambda i: (0, 0)),
            pl.BlockSpec((n_exp, 1), lambda i: (0, 0)),
            pl.BlockSpec((n_top, n_exp), lambda i: (0, 0)),
            pl.BlockSpec((n_top, 1), lambda i: (0, 0)),
        ],
        out_specs=[
            pl.BlockSpec((n_top, _BLOCK), lambda i: (0, i)),
            pl.BlockSpec((_TOP_K, _BLOCK), lambda i: (0, i)),
            pl.BlockSpec((1, 1), lambda i: (0, 0)),
        ],
        out_shape=[
            jax.ShapeDtypeStruct((n_top, n), jnp.float32),
            jax.ShapeDtypeStruct((_TOP_K, n), jnp.int32),
            jax.ShapeDtypeStruct((1, 1), jnp.float32),
        ],
        scratch_shapes=[
            pltpu.VMEM((n_exp, 1), jnp.float32),
            pltpu.VMEM((n_exp, 1), jnp.float32),
        ],
    )(u, W_g.T, b_g.reshape(-1, 1), W_h.T, b_h.reshape(-1, 1))
    return (out_t.T, aux[0, 0], idx_t.T)
